# Initial kernel scaffold; baseline (speedup 1.0000x reference)
import jax
import jax.numpy as jnp
from jax.experimental import pallas as pl


def kernel(logits, prefix_sum):
    raise NotImplementedError("write your pallas kernel here")


# speedup vs baseline: 7.9705x; 7.9705x over previous
import functools

import jax
import jax.numpy as jnp
from jax import lax
from jax.experimental import pallas as pl
from jax.experimental.pallas import tpu as pltpu
from jax.experimental.pallas import tpu_sc as plsc

NC = 2
NS = 16
NW = NC * NS
LANES = 16


def _make_phase1(T, D, B, CH):
    NJ = D // LANES
    RW = T // NW
    NCHUNK = RW // CH
    mesh = plsc.VectorSubcoreMesh(core_axis_name="c", subcore_axis_name="s")

    @functools.partial(
        pl.kernel,
        out_type=(
            jax.ShapeDtypeStruct((NW, B, D), jnp.float32),
            jax.ShapeDtypeStruct((NW, B, D), jnp.float32),
        ),
        mesh=mesh,
        scratch_types=[
            pltpu.VMEM((CH, D), jnp.float32),
            pltpu.VMEM((B, D), jnp.float32),
            pltpu.VMEM((B, D), jnp.float32),
            pltpu.VMEM((32,), jnp.int32),
        ],
    )
    def phase1(x_hbm, ps_hbm, pmax_hbm, psum_hbm, buf, m_v, s_v, ps_s):
        wid = lax.axis_index("s") * NC + lax.axis_index("c")
        lo = wid * RW
        pltpu.sync_copy(ps_hbm, ps_s)
        psv0 = ps_s[pl.ds(0, LANES)]
        psv1 = ps_s[pl.ds(LANES, LANES)]
        ps = [psv0[k] for k in range(LANES)] + [psv1[k] for k in range(LANES)]

        neg = jnp.full((LANES,), -jnp.inf, jnp.float32)
        zero = jnp.zeros((LANES,), jnp.float32)

        def init_body(sb, carry):
            for j in range(NJ):
                m_v[sb, pl.ds(j * LANES, LANES)] = neg
                s_v[sb, pl.ds(j * LANES, LANES)] = zero
            return carry

        lax.fori_loop(0, B, init_body, 0)

        def chunk_body(ci, carry):
            base = lo + ci * CH
            pltpu.sync_copy(x_hbm.at[pl.ds(base, CH), :], buf)
            for sb in range(B):
                r0 = jnp.clip(ps[sb] - base, 0, CH)
                r1 = jnp.clip(ps[sb + 1] - base, 0, CH)

                def amax_body(r, acc):
                    return tuple(
                        jnp.maximum(acc[j], buf[r, pl.ds(j * LANES, LANES)])
                        for j in range(NJ)
                    )

                cmax = lax.fori_loop(r0, r1, amax_body, (neg,) * NJ)

                @pl.when(r1 > r0)
                def _():
                    mnew = []
                    for j in range(NJ):
                        sl = pl.ds(j * LANES, LANES)
                        mo = m_v[sb, sl]
                        mn = jnp.maximum(mo, cmax[j])
                        s_v[sb, sl] = s_v[sb, sl] * jnp.exp(mo - mn)
                        m_v[sb, sl] = mn
                        mnew.append(mn)

                    def bsum_body(r, acc):
                        return tuple(
                            acc[j]
                            + jnp.exp(buf[r, pl.ds(j * LANES, LANES)] - mnew[j])
                            for j in range(NJ)
                        )

                    ssum = lax.fori_loop(r0, r1, bsum_body, (zero,) * NJ)
                    for j in range(NJ):
                        sl = pl.ds(j * LANES, LANES)
                        s_v[sb, sl] = s_v[sb, sl] + ssum[j]

            return carry

        lax.fori_loop(0, NCHUNK, chunk_body, 0)
        pltpu.sync_copy(m_v, pmax_hbm.at[wid])
        pltpu.sync_copy(s_v, psum_hbm.at[wid])

    return phase1


def _combine_body(pm_ref, psm_ref, b_ref):
    pm = pm_ref[...]
    sm = psm_ref[...]
    m = jnp.max(pm, axis=0)
    z = jnp.sum(jnp.where(sm > 0, sm * jnp.exp(pm - m[None]), 0.0), axis=0)
    b_ref[...] = m + jnp.log(z)


def _make_phase3(T, D, B, CH):
    NJ = D // LANES
    RW = T // NW
    NCHUNK = RW // CH
    mesh = plsc.VectorSubcoreMesh(core_axis_name="c", subcore_axis_name="s")

    @functools.partial(
        pl.kernel,
        out_type=jax.ShapeDtypeStruct((T, D), jnp.float32),
        mesh=mesh,
        scratch_types=[
            pltpu.VMEM((CH, D), jnp.float32),
            pltpu.VMEM((B, D), jnp.float32),
            pltpu.VMEM((32,), jnp.int32),
        ],
    )
    def phase3(x_hbm, ps_hbm, b_hbm, out_hbm, buf, b_v, ps_s):
        wid = lax.axis_index("s") * NC + lax.axis_index("c")
        lo = wid * RW
        pltpu.sync_copy(ps_hbm, ps_s)
        pltpu.sync_copy(b_hbm, b_v)
        psv0 = ps_s[pl.ds(0, LANES)]
        psv1 = ps_s[pl.ds(LANES, LANES)]
        ps = [psv0[k] for k in range(LANES)] + [psv1[k] for k in range(LANES)]

        def chunk_body(ci, carry):
            base = lo + ci * CH
            pltpu.sync_copy(x_hbm.at[pl.ds(base, CH), :], buf)
            for sb in range(B):
                r0 = jnp.clip(ps[sb] - base, 0, CH)
                r1 = jnp.clip(ps[sb + 1] - base, 0, CH)
                bj = [b_v[sb, pl.ds(j * LANES, LANES)] for j in range(NJ)]

                def sub_body(r, c):
                    for j in range(NJ):
                        sl = pl.ds(j * LANES, LANES)
                        buf[r, sl] = buf[r, sl] - bj[j]
                    return c

                lax.fori_loop(r0, r1, sub_body, 0)
            pltpu.sync_copy(buf, out_hbm.at[pl.ds(base, CH), :])
            return carry

        lax.fori_loop(0, NCHUNK, chunk_body, 0)

    return phase3


def kernel(logits, prefix_sum):
    T, D = logits.shape
    B = prefix_sum.shape[0] - 1
    CH = 256

    ps_pad = jnp.concatenate(
        [prefix_sum, jnp.full((32 - (B + 1),), T, dtype=prefix_sum.dtype)]
    )

    pm, psm = _make_phase1(T, D, B, CH)(logits, ps_pad)
    b = pl.pallas_call(
        _combine_body,
        out_shape=jax.ShapeDtypeStruct((B, D), jnp.float32),
    )(pm, psm)
    return _make_phase3(T, D, B, CH)(logits, ps_pad, b)


# speedup vs baseline: 10.5981x; 1.3297x over previous
import functools

import jax
import jax.numpy as jnp
from jax import lax
from jax.experimental import pallas as pl
from jax.experimental.pallas import tpu as pltpu
from jax.experimental.pallas import tpu_sc as plsc

NC = 2
NS = 16
NW = NC * NS
LANES = 16


def _stage_offsets(ps_hbm, ps_v, ps_sm, B, T):
    nmem = min(LANES, B)
    pltpu.sync_copy(ps_hbm.at[pl.ds(0, nmem)], ps_v.at[pl.ds(0, nmem)])
    for k0 in range(0, B, LANES):
        v = ps_v[pl.ds(k0, LANES)]
        for k in range(min(LANES, B - k0)):
            ps_sm[k0 + k] = v[k]
    ps_sm[B] = jnp.int32(T)


def _make_phase1(T, D, B, CH):
    NJ = D // LANES
    RW = T // NW
    NCHUNK = RW // CH
    NP = NCHUNK // 2
    assert NCHUNK % 2 == 0
    mesh = plsc.VectorSubcoreMesh(core_axis_name="c", subcore_axis_name="s")

    @functools.partial(
        pl.kernel,
        out_type=(
            jax.ShapeDtypeStruct((NW, B, D), jnp.float32),
            jax.ShapeDtypeStruct((NW, B, D), jnp.float32),
        ),
        mesh=mesh,
        scratch_types=[
            pltpu.VMEM((CH, D), jnp.float32),
            pltpu.VMEM((CH, D), jnp.float32),
            pltpu.VMEM((B, D), jnp.float32),
            pltpu.VMEM((B, D), jnp.float32),
            pltpu.VMEM((LANES,), jnp.int32),
            pltpu.SMEM((32,), jnp.int32),
            pltpu.SemaphoreType.DMA,
            pltpu.SemaphoreType.DMA,
        ],
    )
    def phase1(x_hbm, ps_hbm, pmax_hbm, psum_hbm, bufa, bufb, m_v, s_v, ps_v,
               ps_sm, isem_a, isem_b):
        wid = lax.axis_index("s") * NC + lax.axis_index("c")
        lo = wid * RW

        pltpu.async_copy(x_hbm.at[pl.ds(lo, CH), :], bufa, isem_a)
        if NCHUNK > 1:
            pltpu.async_copy(x_hbm.at[pl.ds(lo + CH, CH), :], bufb, isem_b)

        _stage_offsets(ps_hbm, ps_v, ps_sm, B, T)

        neg = jnp.full((LANES,), -jnp.inf, jnp.float32)
        zero = jnp.zeros((LANES,), jnp.float32)

        def init_body(sb, carry):
            for j in range(NJ):
                m_v[sb, pl.ds(j * LANES, LANES)] = neg
                s_v[sb, pl.ds(j * LANES, LANES)] = zero
            return carry

        lax.fori_loop(0, B, init_body, 0)

        def process(buf, base):
            def seg_body(sb, carry):
                r0 = jnp.clip(ps_sm[sb] - base, 0, CH)
                r1 = jnp.clip(ps_sm[sb + 1] - base, 0, CH)

                @pl.when(r1 > r0)
                def _():
                    @plsc.parallel_loop(r0, r1, unroll=4, carry=(neg,) * NJ)
                    def cmax(r, acc):
                        return tuple(
                            jnp.maximum(acc[j], buf[r, pl.ds(j * LANES, LANES)])
                            for j in range(NJ)
                        )

                    mnew = []
                    for j in range(NJ):
                        sl = pl.ds(j * LANES, LANES)
                        mo = m_v[sb, sl]
                        mn = jnp.maximum(mo, cmax[j])
                        s_v[sb, sl] = s_v[sb, sl] * jnp.exp(mo - mn)
                        m_v[sb, sl] = mn
                        mnew.append(mn)

                    @plsc.parallel_loop(r0, r1, unroll=4, carry=(zero,) * NJ)
                    def ssum(r, acc):
                        return tuple(
                            acc[j]
                            + jnp.exp(buf[r, pl.ds(j * LANES, LANES)] - mnew[j])
                            for j in range(NJ)
                        )

                    for j in range(NJ):
                        sl = pl.ds(j * LANES, LANES)
                        s_v[sb, sl] = s_v[sb, sl] + ssum[j]

                return carry

            lax.fori_loop(0, B, seg_body, 0)

        def pair_body(p, carry):
            base0 = lo + (2 * p) * CH
            pltpu.make_async_copy(
                x_hbm.at[pl.ds(base0, CH), :], bufa, isem_a).wait()
            process(bufa, base0)

            @pl.when(p + 1 < NP)
            def _():
                pltpu.async_copy(
                    x_hbm.at[pl.ds(base0 + 2 * CH, CH), :], bufa, isem_a)

            pltpu.make_async_copy(
                x_hbm.at[pl.ds(base0 + CH, CH), :], bufb, isem_b).wait()
            process(bufb, base0 + CH)

            @pl.when(p + 1 < NP)
            def _():
                pltpu.async_copy(
                    x_hbm.at[pl.ds(base0 + 3 * CH, CH), :], bufb, isem_b)

            return carry

        lax.fori_loop(0, NP, pair_body, 0)

        pltpu.sync_copy(m_v, pmax_hbm.at[wid])
        pltpu.sync_copy(s_v, psum_hbm.at[wid])

    return phase1


def _combine_body(pm_ref, psm_ref, b_ref):
    pm = pm_ref[...]
    sm = psm_ref[...]
    m = jnp.max(pm, axis=0)
    z = jnp.sum(jnp.where(sm > 0, sm * jnp.exp(pm - m[None]), 0.0), axis=0)
    b_ref[...] = m + jnp.log(z)


def _make_phase3(T, D, B, CH):
    NJ = D // LANES
    RW = T // NW
    NCHUNK = RW // CH
    NP = NCHUNK // 2
    assert NCHUNK % 2 == 0
    mesh = plsc.VectorSubcoreMesh(core_axis_name="c", subcore_axis_name="s")

    @functools.partial(
        pl.kernel,
        out_type=jax.ShapeDtypeStruct((T, D), jnp.float32),
        mesh=mesh,
        scratch_types=[
            pltpu.VMEM((CH, D), jnp.float32),
            pltpu.VMEM((CH, D), jnp.float32),
            pltpu.VMEM((B, D), jnp.float32),
            pltpu.VMEM((LANES,), jnp.int32),
            pltpu.SMEM((32,), jnp.int32),
            pltpu.SemaphoreType.DMA,
            pltpu.SemaphoreType.DMA,
            pltpu.SemaphoreType.DMA,
            pltpu.SemaphoreType.DMA,
        ],
    )
    def phase3(x_hbm, ps_hbm, b_hbm, out_hbm, bufa, bufb, b_v, ps_v, ps_sm,
               isem_a, isem_b, osem_a, osem_b):
        wid = lax.axis_index("s") * NC + lax.axis_index("c")
        lo = wid * RW

        pltpu.async_copy(x_hbm.at[pl.ds(lo, CH), :], bufa, isem_a)
        if NCHUNK > 1:
            pltpu.async_copy(x_hbm.at[pl.ds(lo + CH, CH), :], bufb, isem_b)

        _stage_offsets(ps_hbm, ps_v, ps_sm, B, T)
        pltpu.sync_copy(b_hbm, b_v)

        def process(buf, base):
            def seg_body(sb, carry):
                r0 = jnp.clip(ps_sm[sb] - base, 0, CH)
                r1 = jnp.clip(ps_sm[sb + 1] - base, 0, CH)

                @pl.when(r1 > r0)
                def _():
                    bj = [b_v[sb, pl.ds(j * LANES, LANES)] for j in range(NJ)]

                    @plsc.parallel_loop(r0, r1, unroll=4)
                    def _sub(r):
                        for j in range(NJ):
                            sl = pl.ds(j * LANES, LANES)
                            buf[r, sl] = buf[r, sl] - bj[j]

                return carry

            lax.fori_loop(0, B, seg_body, 0)

        def pair_body(p, carry):
            base0 = lo + (2 * p) * CH
            base1 = base0 + CH
            pltpu.make_async_copy(
                x_hbm.at[pl.ds(base0, CH), :], bufa, isem_a).wait()
            process(bufa, base0)
            pltpu.async_copy(bufa, out_hbm.at[pl.ds(base0, CH), :], osem_a)

            pltpu.make_async_copy(
                x_hbm.at[pl.ds(base1, CH), :], bufb, isem_b).wait()
            process(bufb, base1)
            pltpu.async_copy(bufb, out_hbm.at[pl.ds(base1, CH), :], osem_b)

            @pl.when(p + 1 < NP)
            def _():
                pltpu.make_async_copy(
                    bufa, out_hbm.at[pl.ds(base0, CH), :], osem_a).wait()
                pltpu.async_copy(
                    x_hbm.at[pl.ds(base0 + 2 * CH, CH), :], bufa, isem_a)
                pltpu.make_async_copy(
                    bufb, out_hbm.at[pl.ds(base1, CH), :], osem_b).wait()
                pltpu.async_copy(
                    x_hbm.at[pl.ds(base1 + 2 * CH, CH), :], bufb, isem_b)

            return carry

        lax.fori_loop(0, NP, pair_body, 0)

        last0 = lo + (NCHUNK - 2) * CH
        pltpu.make_async_copy(
            bufa, out_hbm.at[pl.ds(last0, CH), :], osem_a).wait()
        pltpu.make_async_copy(
            bufb, out_hbm.at[pl.ds(last0 + CH, CH), :], osem_b).wait()

    return phase3


def kernel(logits, prefix_sum):
    T, D = logits.shape
    B = prefix_sum.shape[0] - 1
    CH = 256

    pm, psm = _make_phase1(T, D, B, CH)(logits, prefix_sum)
    b = pl.pallas_call(
        _combine_body,
        out_shape=jax.ShapeDtypeStruct((B, D), jnp.float32),
    )(pm, psm)
    return _make_phase3(T, D, B, CH)(logits, prefix_sum, b)
